# Initial kernel scaffold; baseline (speedup 1.0000x reference)
import functools

import jax
import jax.numpy as jnp
from jax import lax
from jax.experimental import pallas as pl
from jax.experimental.pallas import tpu as pltpu
from jax.experimental.pallas import tpu_sc as plsc

N = 10000
E = 320000
D = 128

NC = 2
NS = 16
NW = NC * NS
EPW = E // NW
K = 125
C = EPW // K
TPN = N // NS
DW = 16

BM = 2000
G = N // BM


def _mesh():
    return plsc.VectorSubcoreMesh(core_axis_name="c", subcore_axis_name="s")


@functools.partial(
    pl.kernel,
    out_type=jax.ShapeDtypeStruct((NC, N, DW), jnp.float32),
    mesh=_mesh(),
    scratch_types=[
        pltpu.VMEM((C, K), jnp.int32),
        pltpu.VMEM((K, DW), jnp.float32),
        pltpu.VMEM_SHARED((N, DW), jnp.float32),
    ],
)
def _deg_kernel(dst_hbm, zeros_hbm, ones_hbm, out_hbm, dst_v, ones_v, acc):
    cid = lax.axis_index("c")
    sid = lax.axis_index("s")
    wid = cid * NS + sid
    pltpu.sync_copy(zeros_hbm, acc.at[pl.ds(sid * TPN, TPN)])
    pltpu.sync_copy(dst_hbm.at[wid], dst_v)
    pltpu.sync_copy(ones_hbm, ones_v)
    plsc.subcore_barrier()

    def body(j, carry):
        pltpu.sync_copy(ones_v, acc.at[dst_v.at[j]], add=True)
        return carry

    lax.fori_loop(0, C, body, 0)
    plsc.subcore_barrier()
    pltpu.sync_copy(acc.at[pl.ds(sid * TPN, TPN)],
                    out_hbm.at[cid, pl.ds(sid * TPN, TPN)])


@functools.partial(
    pl.kernel,
    out_type=jax.ShapeDtypeStruct((NC, N, D), jnp.float32),
    mesh=_mesh(),
    scratch_types=[
        pltpu.VMEM((C, K), jnp.int32),
        pltpu.VMEM((C, K), jnp.int32),
        pltpu.VMEM((K, D), jnp.float32),
        pltpu.VMEM_SHARED((N, D), jnp.float32),
        pltpu.SemaphoreType.DMA,
    ],
)
def _scatter_kernel(y_hbm, src_hbm, dst_hbm, zeros_hbm, out_hbm,
                    src_v, dst_v, rows_v, acc, sem):
    cid = lax.axis_index("c")
    sid = lax.axis_index("s")
    wid = cid * NS + sid
    pltpu.sync_copy(zeros_hbm, acc.at[pl.ds(sid * TPN, TPN)])
    pltpu.sync_copy(src_hbm.at[wid], src_v)
    pltpu.sync_copy(dst_hbm.at[wid], dst_v)
    plsc.subcore_barrier()

    def body(j, carry):
        pltpu.async_copy(y_hbm.at[src_v.at[j]], rows_v, sem).wait()
        pltpu.sync_copy(rows_v, acc.at[dst_v.at[j]], add=True)
        return carry

    lax.fori_loop(0, C, body, 0)
    plsc.subcore_barrier()
    pltpu.sync_copy(acc.at[pl.ds(sid * TPN, TPN)],
                    out_hbm.at[cid, pl.ds(sid * TPN, TPN)])


def _dis_block(deg_ref):
    deg = jnp.sum(deg_ref[...], axis=2)
    deg = deg[0] + deg[1] + 1.0
    return lax.rsqrt(deg)


def _mm_body(x_ref, w_ref, o_ref):
    o_ref[...] = jnp.dot(x_ref[...], w_ref[...],
                         preferred_element_type=jnp.float32)


def _scale_body(deg_ref, xw_ref, o_ref):
    dis = _dis_block(deg_ref)
    o_ref[...] = xw_ref[...] * dis[:, None]


def _layer2_body(deg_ref, agg_ref, y1_ref, b1_ref, w2_ref, o_ref):
    dis = _dis_block(deg_ref)
    tot = agg_ref[0] + agg_ref[1] + y1_ref[...]
    h = jnp.maximum(tot * dis[:, None] + b1_ref[...], 0.0)
    o_ref[...] = jnp.dot(h, w2_ref[...],
                         preferred_element_type=jnp.float32) * dis[:, None]


def _final_body(deg_ref, agg_ref, y2_ref, b2_ref, o_ref):
    dis = _dis_block(deg_ref)
    tot = agg_ref[0] + agg_ref[1] + y2_ref[...]
    o_ref[...] = tot * dis[:, None] + b2_ref[...]


def _row_spec():
    return pl.BlockSpec((BM, D), lambda i: (i, 0))


def _deg_spec():
    return pl.BlockSpec((NC, BM, DW), lambda i: (0, i, 0))


def _agg_spec():
    return pl.BlockSpec((NC, BM, D), lambda i: (0, i, 0))


def _full_spec(shape):
    nd = len(shape)
    return pl.BlockSpec(shape, lambda i: (0,) * nd)


def _matmul(x, W):
    return pl.pallas_call(
        _mm_body,
        grid=(G,),
        in_specs=[_row_spec(), _full_spec((D, D))],
        out_specs=_row_spec(),
        out_shape=jax.ShapeDtypeStruct((N, D), jnp.float32),
    )(x, W)


def _scale(deg_parts, xw):
    return pl.pallas_call(
        _scale_body,
        grid=(G,),
        in_specs=[_deg_spec(), _row_spec()],
        out_specs=_row_spec(),
        out_shape=jax.ShapeDtypeStruct((N, D), jnp.float32),
    )(deg_parts, xw)


def _layer2(deg_parts, agg, y1, b1, W2):
    return pl.pallas_call(
        _layer2_body,
        grid=(G,),
        in_specs=[_deg_spec(), _agg_spec(), _row_spec(),
                  _full_spec((1, D)), _full_spec((D, D))],
        out_specs=_row_spec(),
        out_shape=jax.ShapeDtypeStruct((N, D), jnp.float32),
    )(deg_parts, agg, y1, b1, W2)


def _final(deg_parts, agg, y2, b2):
    return pl.pallas_call(
        _final_body,
        grid=(G,),
        in_specs=[_deg_spec(), _agg_spec(), _row_spec(), _full_spec((1, D))],
        out_specs=_row_spec(),
        out_shape=jax.ShapeDtypeStruct((N, D), jnp.float32),
    )(deg_parts, agg, y2, b2)


def kernel(x, edge_index, W1, b1, W2, b2):
    src = edge_index[0].astype(jnp.int32).reshape(NW, C, K)
    dst = edge_index[1].astype(jnp.int32).reshape(NW, C, K)
    zeros_deg = jnp.zeros((TPN, DW), jnp.float32)
    onehot = jnp.zeros((K, DW), jnp.float32).at[:, 0].set(1.0)
    zeros_row = jnp.zeros((TPN, D), jnp.float32)

    deg_parts = _deg_kernel(dst, zeros_deg, onehot)
    xw1 = _matmul(x, W1)
    y1 = _scale(deg_parts, xw1)
    agg1 = _scatter_kernel(y1, src, dst, zeros_row)
    y2 = _layer2(deg_parts, agg1, y1, b1.reshape(1, D), W2)
    agg2 = _scatter_kernel(y2, src, dst, zeros_row)
    return _final(deg_parts, agg2, y2, b2.reshape(1, D))


# speedup vs baseline: 23.8835x; 23.8835x over previous
import functools

import jax
import jax.numpy as jnp
from jax import lax
from jax.experimental import pallas as pl
from jax.experimental.pallas import tpu as pltpu
from jax.experimental.pallas import tpu_sc as plsc

N = 10000
E = 320000
D = 128

NC = 2
NS = 16
NW = NC * NS
EPW = E // NW
K = 125
C = EPW // K
NP = 10240
TPN = NP // NS
DW = 16

BM = 2048
G = (N + BM - 1) // BM


def _mesh():
    return plsc.VectorSubcoreMesh(core_axis_name="c", subcore_axis_name="s")


@functools.partial(
    pl.kernel,
    out_type=jax.ShapeDtypeStruct((NW, NP), jnp.float32),
    mesh=_mesh(),
    compiler_params=pltpu.CompilerParams(needs_layout_passes=False),
    scratch_types=[
        pltpu.VMEM((EPW,), jnp.int32),
        pltpu.VMEM((NP,), jnp.float32),
    ],
)
def _deg_kernel(dst_hbm, zeros_hbm, out_hbm, dst_v, acc):
    cid = lax.axis_index("c")
    sid = lax.axis_index("s")
    wid = cid * NS + sid
    pltpu.sync_copy(zeros_hbm, acc)
    pltpu.sync_copy(dst_hbm.at[wid], dst_v)
    ones = jnp.full((16,), 1.0, jnp.float32)

    def body(j, carry):
        idx = dst_v[pl.ds(j * 16, 16)]
        plsc.addupdate_scatter(acc, [idx], ones)
        return carry

    lax.fori_loop(0, EPW // 16, body, 0)
    pltpu.sync_copy(acc, out_hbm.at[wid])


@functools.partial(
    pl.kernel,
    out_type=jax.ShapeDtypeStruct((NC, NP, D), jnp.float32),
    mesh=_mesh(),
    scratch_types=[
        pltpu.VMEM((C, K), jnp.int32),
        pltpu.VMEM((C, K), jnp.int32),
        pltpu.VMEM((K, D), jnp.float32),
        pltpu.VMEM_SHARED((NP, D), jnp.float32),
        pltpu.SemaphoreType.DMA,
    ],
)
def _scatter_kernel(y_hbm, src_hbm, dst_hbm, zeros_hbm, out_hbm,
                    src_v, dst_v, rows_v, acc, sem):
    cid = lax.axis_index("c")
    sid = lax.axis_index("s")
    wid = cid * NS + sid
    pltpu.sync_copy(zeros_hbm, acc.at[pl.ds(sid * TPN, TPN)])
    pltpu.sync_copy(src_hbm.at[wid], src_v)
    pltpu.sync_copy(dst_hbm.at[wid], dst_v)
    plsc.subcore_barrier()

    def body(j, carry):
        pltpu.async_copy(y_hbm.at[src_v.at[j]], rows_v, sem).wait()
        pltpu.sync_copy(rows_v, acc.at[dst_v.at[j]], add=True)
        return carry

    lax.fori_loop(0, C, body, 0)
    plsc.subcore_barrier()
    pltpu.sync_copy(acc.at[pl.ds(sid * TPN, TPN)],
                    out_hbm.at[cid, pl.ds(sid * TPN, TPN)])


def _dis_block(deg_ref):
    i = pl.program_id(0)
    blk = deg_ref[:, pl.ds(i * BM, BM)]
    deg = jnp.sum(blk, axis=0) + 1.0
    return lax.rsqrt(deg)


def _mm_body(x_ref, w_ref, o_ref):
    o_ref[...] = jnp.dot(x_ref[...], w_ref[...],
                         preferred_element_type=jnp.float32)


def _scale_body(deg_ref, xw_ref, o_ref):
    dis = _dis_block(deg_ref)
    o_ref[...] = xw_ref[...] * dis[:, None]


def _layer2_body(deg_ref, agg_ref, y1_ref, b1_ref, w2_ref, o_ref):
    dis = _dis_block(deg_ref)
    tot = agg_ref[0] + agg_ref[1] + y1_ref[...]
    h = jnp.maximum(tot * dis[:, None] + b1_ref[...], 0.0)
    o_ref[...] = jnp.dot(h, w2_ref[...],
                         preferred_element_type=jnp.float32) * dis[:, None]


def _final_body(deg_ref, agg_ref, y2_ref, b2_ref, o_ref):
    dis = _dis_block(deg_ref)
    tot = agg_ref[0] + agg_ref[1] + y2_ref[...]
    o_ref[...] = tot * dis[:, None] + b2_ref[...]


def _row_spec():
    return pl.BlockSpec((BM, D), lambda i: (i, 0))


def _deg_spec():
    return pl.BlockSpec((NW, NP), lambda i: (0, 0))


def _agg_spec():
    return pl.BlockSpec((NC, BM, D), lambda i: (0, i, 0))


def _full_spec(shape):
    nd = len(shape)
    return pl.BlockSpec(shape, lambda i: (0,) * nd)


def _matmul(x, W):
    return pl.pallas_call(
        _mm_body,
        grid=(G,),
        in_specs=[_row_spec(), _full_spec((D, D))],
        out_specs=_row_spec(),
        out_shape=jax.ShapeDtypeStruct((N, D), jnp.float32),
    )(x, W)


def _scale(deg_parts, xw):
    return pl.pallas_call(
        _scale_body,
        grid=(G,),
        in_specs=[_deg_spec(), _row_spec()],
        out_specs=_row_spec(),
        out_shape=jax.ShapeDtypeStruct((N, D), jnp.float32),
    )(deg_parts, xw)


def _layer2(deg_parts, agg, y1, b1, W2):
    return pl.pallas_call(
        _layer2_body,
        grid=(G,),
        in_specs=[_deg_spec(), _agg_spec(), _row_spec(),
                  _full_spec((1, D)), _full_spec((D, D))],
        out_specs=_row_spec(),
        out_shape=jax.ShapeDtypeStruct((N, D), jnp.float32),
    )(deg_parts, agg, y1, b1, W2)


def _final(deg_parts, agg, y2, b2):
    return pl.pallas_call(
        _final_body,
        grid=(G,),
        in_specs=[_deg_spec(), _agg_spec(), _row_spec(), _full_spec((1, D))],
        out_specs=_row_spec(),
        out_shape=jax.ShapeDtypeStruct((N, D), jnp.float32),
    )(deg_parts, agg, y2, b2)


def kernel(x, edge_index, W1, b1, W2, b2):
    src = edge_index[0].astype(jnp.int32).reshape(NW, C, K)
    dst = edge_index[1].astype(jnp.int32).reshape(NW, C, K)
    dst_flat = edge_index[1].astype(jnp.int32).reshape(NW, EPW)
    zeros_deg = jnp.zeros((NP,), jnp.float32)
    zeros_row = jnp.zeros((TPN, D), jnp.float32)

    deg_parts = _deg_kernel(dst_flat, zeros_deg)
    xw1 = _matmul(x, W1)
    y1 = _scale(deg_parts, xw1)
    agg1 = _scatter_kernel(y1, src, dst, zeros_row)
    y2 = _layer2(deg_parts, agg1, y1, b1.reshape(1, D), W2)
    agg2 = _scatter_kernel(y2, src, dst, zeros_row)
    return _final(deg_parts, agg2, y2, b2.reshape(1, D))


# speedup vs baseline: 35.1845x; 1.4732x over previous
import functools

import jax
import jax.numpy as jnp
from jax import lax
from jax.experimental import pallas as pl
from jax.experimental.pallas import tpu as pltpu
from jax.experimental.pallas import tpu_sc as plsc

N = 10000
E = 320000
D = 128

NC = 2
NS = 16
NW = NC * NS
EPW = E // NW
K = 125
C = EPW // K
NP = 10240
TPN = NP // NS
DW = 16

BM = 2048
G = (N + BM - 1) // BM


def _mesh():
    return plsc.VectorSubcoreMesh(core_axis_name="c", subcore_axis_name="s")


@functools.partial(
    pl.kernel,
    out_type=jax.ShapeDtypeStruct((NW, NP), jnp.float32),
    mesh=_mesh(),
    compiler_params=pltpu.CompilerParams(needs_layout_passes=False),
    scratch_types=[
        pltpu.VMEM((EPW,), jnp.int32),
        pltpu.VMEM((NP,), jnp.float32),
    ],
)
def _deg_kernel(dst_hbm, zeros_hbm, out_hbm, dst_v, acc):
    cid = lax.axis_index("c")
    sid = lax.axis_index("s")
    wid = cid * NS + sid
    pltpu.sync_copy(zeros_hbm, acc)
    pltpu.sync_copy(dst_hbm.at[wid], dst_v)
    ones = jnp.full((16,), 1.0, jnp.float32)

    def body(j, carry):
        idx = dst_v[pl.ds(j * 16, 16)]
        plsc.addupdate_scatter(acc, [idx], ones)
        return carry

    lax.fori_loop(0, EPW // 16, body, 0)
    pltpu.sync_copy(acc, out_hbm.at[wid])


@functools.partial(
    pl.kernel,
    out_type=jax.ShapeDtypeStruct((NC, NP, D), jnp.float32),
    mesh=_mesh(),
    scratch_types=[
        pltpu.VMEM((2, K), jnp.int32),
        pltpu.VMEM((2, K), jnp.int32),
        pltpu.VMEM((2, K), jnp.int32),
        pltpu.VMEM((2, K), jnp.int32),
        pltpu.VMEM((K, D), jnp.float32),
        pltpu.VMEM((K, D), jnp.float32),
        pltpu.VMEM_SHARED((NP, D), jnp.float32),
        pltpu.SemaphoreType.DMA,
        pltpu.SemaphoreType.DMA,
        pltpu.SemaphoreType.DMA,
        pltpu.SemaphoreType.DMA,
        pltpu.SemaphoreType.DMA,
        pltpu.SemaphoreType.DMA,
    ],
)
def _scatter_kernel(y_hbm, sd_hbm, zeros_hbm, out_hbm,
                    idx0, idx1, idx2, idx3, rows0, rows1, acc,
                    isem0, isem1, isem2, isem3, rsem0, rsem1):
    cid = lax.axis_index("c")
    sid = lax.axis_index("s")
    wid = cid * NS + sid
    idx = (idx0, idx1, idx2, idx3)
    isem = (isem0, isem1, isem2, isem3)
    rows = (rows0, rows1)
    rsem = (rsem0, rsem1)

    pltpu.sync_copy(zeros_hbm, acc.at[pl.ds(sid * TPN, TPN)])
    for i in range(4):
        pltpu.async_copy(sd_hbm.at[wid, i], idx[i], isem[i])
    for j in range(2):
        pltpu.make_async_copy(sd_hbm.at[wid, j], idx[j], isem[j]).wait()
        pltpu.async_copy(y_hbm.at[idx[j].at[0]], rows[j], rsem[j])
    plsc.subcore_barrier()

    def body(g, carry):
        for u in range(4):
            j = 4 * g + u
            rb = u % 2
            pltpu.make_async_copy(y_hbm.at[idx[u].at[0]], rows[rb],
                                  rsem[rb]).wait()
            pltpu.sync_copy(rows[rb], acc.at[idx[u].at[1]], add=True)

            @pl.when(j + 4 < C)
            def _():
                pltpu.async_copy(sd_hbm.at[wid, j + 4], idx[u], isem[u])

            @pl.when(j + 2 < C)
            def _():
                u2 = (u + 2) % 4
                pltpu.make_async_copy(sd_hbm.at[wid, j + 2], idx[u2],
                                      isem[u2]).wait()
                pltpu.async_copy(y_hbm.at[idx[u2].at[0]], rows[rb], rsem[rb])
        return carry

    lax.fori_loop(0, C // 4, body, 0)
    plsc.subcore_barrier()
    pltpu.sync_copy(acc.at[pl.ds(sid * TPN, TPN)],
                    out_hbm.at[cid, pl.ds(sid * TPN, TPN)])


def _dis_block(deg_ref):
    i = pl.program_id(0)
    blk = deg_ref[:, pl.ds(i * BM, BM)]
    deg = jnp.sum(blk, axis=0) + 1.0
    return lax.rsqrt(deg)


def _mm_body(x_ref, w_ref, o_ref):
    o_ref[...] = jnp.dot(x_ref[...], w_ref[...],
                         preferred_element_type=jnp.float32)


def _scale_body(deg_ref, xw_ref, o_ref):
    dis = _dis_block(deg_ref)
    o_ref[...] = xw_ref[...] * dis[:, None]


def _layer2_body(deg_ref, agg_ref, y1_ref, b1_ref, w2_ref, o_ref):
    dis = _dis_block(deg_ref)
    tot = agg_ref[0] + agg_ref[1] + y1_ref[...]
    h = jnp.maximum(tot * dis[:, None] + b1_ref[...], 0.0)
    o_ref[...] = jnp.dot(h, w2_ref[...],
                         preferred_element_type=jnp.float32) * dis[:, None]


def _final_body(deg_ref, agg_ref, y2_ref, b2_ref, o_ref):
    dis = _dis_block(deg_ref)
    tot = agg_ref[0] + agg_ref[1] + y2_ref[...]
    o_ref[...] = tot * dis[:, None] + b2_ref[...]


def _row_spec():
    return pl.BlockSpec((BM, D), lambda i: (i, 0))


def _deg_spec():
    return pl.BlockSpec((NW, NP), lambda i: (0, 0))


def _agg_spec():
    return pl.BlockSpec((NC, BM, D), lambda i: (0, i, 0))


def _full_spec(shape):
    nd = len(shape)
    return pl.BlockSpec(shape, lambda i: (0,) * nd)


def _matmul(x, W):
    return pl.pallas_call(
        _mm_body,
        grid=(G,),
        in_specs=[_row_spec(), _full_spec((D, D))],
        out_specs=_row_spec(),
        out_shape=jax.ShapeDtypeStruct((N, D), jnp.float32),
    )(x, W)


def _scale(deg_parts, xw):
    return pl.pallas_call(
        _scale_body,
        grid=(G,),
        in_specs=[_deg_spec(), _row_spec()],
        out_specs=_row_spec(),
        out_shape=jax.ShapeDtypeStruct((N, D), jnp.float32),
    )(deg_parts, xw)


def _layer2(deg_parts, agg, y1, b1, W2):
    return pl.pallas_call(
        _layer2_body,
        grid=(G,),
        in_specs=[_deg_spec(), _agg_spec(), _row_spec(),
                  _full_spec((1, D)), _full_spec((D, D))],
        out_specs=_row_spec(),
        out_shape=jax.ShapeDtypeStruct((N, D), jnp.float32),
    )(deg_parts, agg, y1, b1, W2)


def _final(deg_parts, agg, y2, b2):
    return pl.pallas_call(
        _final_body,
        grid=(G,),
        in_specs=[_deg_spec(), _agg_spec(), _row_spec(), _full_spec((1, D))],
        out_specs=_row_spec(),
        out_shape=jax.ShapeDtypeStruct((N, D), jnp.float32),
    )(deg_parts, agg, y2, b2)


def kernel(x, edge_index, W1, b1, W2, b2):
    src = edge_index[0].astype(jnp.int32).reshape(NW, C, 1, K)
    dst = edge_index[1].astype(jnp.int32).reshape(NW, C, 1, K)
    sd = jnp.concatenate([src, dst], axis=2)
    dst_flat = edge_index[1].astype(jnp.int32).reshape(NW, EPW)
    zeros_deg = jnp.zeros((NP,), jnp.float32)
    zeros_row = jnp.zeros((TPN, D), jnp.float32)

    deg_parts = _deg_kernel(dst_flat, zeros_deg)
    xw1 = _matmul(x, W1)
    y1 = _scale(deg_parts, xw1)
    agg1 = _scatter_kernel(y1, sd, zeros_row)
    y2 = _layer2(deg_parts, agg1, y1, b1.reshape(1, D), W2)
    agg2 = _scatter_kernel(y2, sd, zeros_row)
    return _final(deg_parts, agg2, y2, b2.reshape(1, D))


# speedup vs baseline: 35.8908x; 1.0201x over previous
import functools

import jax
import jax.numpy as jnp
from jax import lax
from jax.experimental import pallas as pl
from jax.experimental.pallas import tpu as pltpu
from jax.experimental.pallas import tpu_sc as plsc

N = 10000
E = 320000
D = 128

NC = 2
NS = 16
NW = NC * NS
EPW = E // NW
K = 125
C = EPW // K
NP = 10240
TPN = NP // NS
DW = 16

BM = 2048
G = (N + BM - 1) // BM


def _mesh():
    return plsc.VectorSubcoreMesh(core_axis_name="c", subcore_axis_name="s")


@functools.partial(
    pl.kernel,
    out_type=jax.ShapeDtypeStruct((NW, NP), jnp.float32),
    mesh=_mesh(),
    compiler_params=pltpu.CompilerParams(needs_layout_passes=False),
    scratch_types=[
        pltpu.VMEM((EPW,), jnp.int32),
        pltpu.VMEM((NP,), jnp.float32),
    ],
)
def _deg_kernel(dst_hbm, zeros_hbm, out_hbm, dst_v, acc):
    cid = lax.axis_index("c")
    sid = lax.axis_index("s")
    wid = cid * NS + sid
    pltpu.sync_copy(zeros_hbm, acc)
    pltpu.sync_copy(dst_hbm.at[wid], dst_v)
    ones = jnp.full((16,), 1.0, jnp.float32)

    def body(j, carry):
        idx = dst_v[pl.ds(j * 16, 16)]
        plsc.addupdate_scatter(acc, [idx], ones)
        return carry

    lax.fori_loop(0, EPW // 16, body, 0)
    pltpu.sync_copy(acc, out_hbm.at[wid])


@functools.partial(
    pl.kernel,
    out_type=jax.ShapeDtypeStruct((NC, NP, D), jnp.float32),
    mesh=_mesh(),
    scratch_types=[
        pltpu.VMEM((2, K), jnp.int32),
        pltpu.VMEM((2, K), jnp.int32),
        pltpu.VMEM((2, K), jnp.int32),
        pltpu.VMEM((2, K), jnp.int32),
        pltpu.VMEM((K, D), jnp.float32),
        pltpu.VMEM((K, D), jnp.float32),
        pltpu.VMEM_SHARED((NP, D), jnp.float32),
        pltpu.SemaphoreType.DMA,
        pltpu.SemaphoreType.DMA,
        pltpu.SemaphoreType.DMA,
        pltpu.SemaphoreType.DMA,
        pltpu.SemaphoreType.DMA,
        pltpu.SemaphoreType.DMA,
    ],
)
def _scatter_kernel(y_hbm, sd_hbm, zeros_hbm, out_hbm,
                    idx0, idx1, idx2, idx3, rows0, rows1, acc,
                    isem0, isem1, isem2, isem3, rsem0, rsem1):
    cid = lax.axis_index("c")
    sid = lax.axis_index("s")
    wid = cid * NS + sid
    idx = (idx0, idx1, idx2, idx3)
    isem = (isem0, isem1, isem2, isem3)
    rows = (rows0, rows1)
    rsem = (rsem0, rsem1)

    for i in range(4):
        pltpu.async_copy(sd_hbm.at[wid, i], idx[i], isem[i])
    pltpu.sync_copy(zeros_hbm, acc.at[pl.ds(sid * TPN, TPN)])
    for j in range(2):
        pltpu.make_async_copy(sd_hbm.at[wid, j], idx[j], isem[j]).wait()
        pltpu.async_copy(y_hbm.at[idx[j].at[0]], rows[j], rsem[j])
    plsc.subcore_barrier()

    def body(g, carry):
        for u in range(4):
            j = 4 * g + u
            rb = u % 2
            pltpu.make_async_copy(y_hbm.at[idx[u].at[0]], rows[rb],
                                  rsem[rb]).wait()
            pltpu.sync_copy(rows[rb], acc.at[idx[u].at[1]], add=True)

            @pl.when(j + 4 < C)
            def _():
                pltpu.async_copy(sd_hbm.at[wid, j + 4], idx[u], isem[u])

            @pl.when(j + 2 < C)
            def _():
                u2 = (u + 2) % 4
                pltpu.make_async_copy(sd_hbm.at[wid, j + 2], idx[u2],
                                      isem[u2]).wait()
                pltpu.async_copy(y_hbm.at[idx[u2].at[0]], rows[rb], rsem[rb])
        return carry

    lax.fori_loop(0, C // 4, body, 0)
    plsc.subcore_barrier()
    pltpu.sync_copy(acc.at[pl.ds(sid * TPN, TPN)],
                    out_hbm.at[cid, pl.ds(sid * TPN, TPN)])


def _dis_block(deg_ref):
    i = pl.program_id(0)
    blk = deg_ref[:, pl.ds(i * BM, BM)]
    deg = jnp.sum(blk, axis=0) + 1.0
    return lax.rsqrt(deg)


def _mm_scale_body(deg_ref, x_ref, w_ref, o_ref):
    dis = _dis_block(deg_ref)
    o_ref[...] = jnp.dot(x_ref[...], w_ref[...],
                         preferred_element_type=jnp.float32) * dis[:, None]


def _layer2_body(deg_ref, agg_ref, y1_ref, b1_ref, w2_ref, o_ref):
    dis = _dis_block(deg_ref)
    tot = agg_ref[0] + agg_ref[1] + y1_ref[...]
    h = jnp.maximum(tot * dis[:, None] + b1_ref[...], 0.0)
    o_ref[...] = jnp.dot(h, w2_ref[...],
                         preferred_element_type=jnp.float32) * dis[:, None]


def _final_body(deg_ref, agg_ref, y2_ref, b2_ref, o_ref):
    dis = _dis_block(deg_ref)
    tot = agg_ref[0] + agg_ref[1] + y2_ref[...]
    o_ref[...] = tot * dis[:, None] + b2_ref[...]


def _row_spec():
    return pl.BlockSpec((BM, D), lambda i: (i, 0))


def _deg_spec():
    return pl.BlockSpec((NW, NP), lambda i: (0, 0))


def _agg_spec():
    return pl.BlockSpec((NC, BM, D), lambda i: (0, i, 0))


def _full_spec(shape):
    nd = len(shape)
    return pl.BlockSpec(shape, lambda i: (0,) * nd)


def _mm_scale(deg_parts, x, W):
    return pl.pallas_call(
        _mm_scale_body,
        grid=(G,),
        in_specs=[_deg_spec(), _row_spec(), _full_spec((D, D))],
        out_specs=_row_spec(),
        out_shape=jax.ShapeDtypeStruct((N, D), jnp.float32),
    )(deg_parts, x, W)


def _layer2(deg_parts, agg, y1, b1, W2):
    return pl.pallas_call(
        _layer2_body,
        grid=(G,),
        in_specs=[_deg_spec(), _agg_spec(), _row_spec(),
                  _full_spec((1, D)), _full_spec((D, D))],
        out_specs=_row_spec(),
        out_shape=jax.ShapeDtypeStruct((N, D), jnp.float32),
    )(deg_parts, agg, y1, b1, W2)


def _final(deg_parts, agg, y2, b2):
    return pl.pallas_call(
        _final_body,
        grid=(G,),
        in_specs=[_deg_spec(), _agg_spec(), _row_spec(), _full_spec((1, D))],
        out_specs=_row_spec(),
        out_shape=jax.ShapeDtypeStruct((N, D), jnp.float32),
    )(deg_parts, agg, y2, b2)


def kernel(x, edge_index, W1, b1, W2, b2):
    src = edge_index[0].astype(jnp.int32).reshape(NW, C, 1, K)
    dst = edge_index[1].astype(jnp.int32).reshape(NW, C, 1, K)
    sd = jnp.concatenate([src, dst], axis=2)
    dst_flat = edge_index[1].astype(jnp.int32).reshape(NW, EPW)
    zeros_deg = jnp.zeros((NP,), jnp.float32)
    zeros_row = jnp.zeros((TPN, D), jnp.float32)

    deg_parts = _deg_kernel(dst_flat, zeros_deg)
    y1 = _mm_scale(deg_parts, x, W1)
    agg1 = _scatter_kernel(y1, sd, zeros_row)
    y2 = _layer2(deg_parts, agg1, y1, b1.reshape(1, D), W2)
    agg2 = _scatter_kernel(y2, sd, zeros_row)
    return _final(deg_parts, agg2, y2, b2.reshape(1, D))


# speedup vs baseline: 36.7013x; 1.0226x over previous
import functools

import jax
import jax.numpy as jnp
from jax import lax
from jax.experimental import pallas as pl
from jax.experimental.pallas import tpu as pltpu
from jax.experimental.pallas import tpu_sc as plsc

N = 10000
E = 320000
D = 128

NC = 2
NS = 16
NW = NC * NS
EPW = E // NW
K = 125
C = EPW // K
NP = 10240
TPN = NP // NS
DW = 16

BM = 2048
G = (N + BM - 1) // BM


def _mesh():
    return plsc.VectorSubcoreMesh(core_axis_name="c", subcore_axis_name="s")


@functools.partial(
    pl.kernel,
    out_type=jax.ShapeDtypeStruct((NW, NP), jnp.float32),
    mesh=_mesh(),
    compiler_params=pltpu.CompilerParams(needs_layout_passes=False),
    scratch_types=[
        pltpu.VMEM((EPW,), jnp.int32),
        pltpu.VMEM((NP,), jnp.float32),
    ],
)
def _deg_kernel(dst_hbm, zeros_hbm, out_hbm, dst_v, acc):
    cid = lax.axis_index("c")
    sid = lax.axis_index("s")
    wid = cid * NS + sid
    pltpu.sync_copy(zeros_hbm, acc)
    pltpu.sync_copy(dst_hbm.at[wid], dst_v)
    ones = jnp.full((16,), 1.0, jnp.float32)

    def body(j, carry):
        idx = dst_v[pl.ds(j * 16, 16)]
        plsc.addupdate_scatter(acc, [idx], ones)
        return carry

    lax.fori_loop(0, EPW // 16, body, 0)
    pltpu.sync_copy(acc, out_hbm.at[wid])


@functools.partial(
    pl.kernel,
    out_type=jax.ShapeDtypeStruct((NC, NP, D), jnp.float32),
    mesh=_mesh(),
    scratch_types=[
        pltpu.VMEM((2, K), jnp.int32),
        pltpu.VMEM((2, K), jnp.int32),
        pltpu.VMEM((2, K), jnp.int32),
        pltpu.VMEM((2, K), jnp.int32),
        pltpu.VMEM((K, D), jnp.float32),
        pltpu.VMEM((K, D), jnp.float32),
        pltpu.VMEM_SHARED((NP, D), jnp.float32),
        pltpu.SemaphoreType.DMA,
        pltpu.SemaphoreType.DMA,
        pltpu.SemaphoreType.DMA,
        pltpu.SemaphoreType.DMA,
        pltpu.SemaphoreType.DMA,
        pltpu.SemaphoreType.DMA,
    ],
)
def _scatter_kernel(y_hbm, sd_hbm, zeros_hbm, out_hbm,
                    idx0, idx1, idx2, idx3, rows0, rows1, acc,
                    isem0, isem1, isem2, isem3, rsem0, rsem1):
    cid = lax.axis_index("c")
    sid = lax.axis_index("s")
    wid = cid * NS + sid
    idx = (idx0, idx1, idx2, idx3)
    isem = (isem0, isem1, isem2, isem3)
    rows = (rows0, rows1)
    rsem = (rsem0, rsem1)

    for i in range(4):
        pltpu.async_copy(sd_hbm.at[wid, i], idx[i], isem[i])

    @pl.when(cid == 0)
    def _():
        pltpu.sync_copy(y_hbm.at[pl.ds(sid * TPN, TPN)],
                        acc.at[pl.ds(sid * TPN, TPN)])

    @pl.when(cid != 0)
    def _():
        pltpu.sync_copy(zeros_hbm, acc.at[pl.ds(sid * TPN, TPN)])
    for j in range(2):
        pltpu.make_async_copy(sd_hbm.at[wid, j], idx[j], isem[j]).wait()
        pltpu.async_copy(y_hbm.at[idx[j].at[0]], rows[j], rsem[j])
    plsc.subcore_barrier()

    def body(g, carry):
        for u in range(4):
            j = 4 * g + u
            rb = u % 2
            pltpu.make_async_copy(y_hbm.at[idx[u].at[0]], rows[rb],
                                  rsem[rb]).wait()
            pltpu.sync_copy(rows[rb], acc.at[idx[u].at[1]], add=True)

            @pl.when(j + 4 < C)
            def _():
                pltpu.async_copy(sd_hbm.at[wid, j + 4], idx[u], isem[u])

            @pl.when(j + 2 < C)
            def _():
                u2 = (u + 2) % 4
                pltpu.make_async_copy(sd_hbm.at[wid, j + 2], idx[u2],
                                      isem[u2]).wait()
                pltpu.async_copy(y_hbm.at[idx[u2].at[0]], rows[rb], rsem[rb])
        return carry

    lax.fori_loop(0, C // 4, body, 0)
    plsc.subcore_barrier()
    pltpu.sync_copy(acc.at[pl.ds(sid * TPN, TPN)],
                    out_hbm.at[cid, pl.ds(sid * TPN, TPN)])


def _dis_block(deg_ref):
    i = pl.program_id(0)
    blk = deg_ref[:, pl.ds(i * BM, BM)]
    deg = jnp.sum(blk, axis=0) + 1.0
    return lax.rsqrt(deg)


def _mm_scale_body(deg_ref, x_ref, w_ref, o_ref):
    dis = _dis_block(deg_ref)
    o_ref[...] = jnp.dot(x_ref[...], w_ref[...],
                         preferred_element_type=jnp.float32) * dis[:, None]


def _layer2_body(deg_ref, agg_ref, b1_ref, w2_ref, o_ref):
    dis = _dis_block(deg_ref)
    tot = agg_ref[0] + agg_ref[1]
    h = jnp.maximum(tot * dis[:, None] + b1_ref[...], 0.0)
    o_ref[...] = jnp.dot(h, w2_ref[...],
                         preferred_element_type=jnp.float32) * dis[:, None]


def _final_body(deg_ref, agg_ref, b2_ref, o_ref):
    dis = _dis_block(deg_ref)
    tot = agg_ref[0] + agg_ref[1]
    o_ref[...] = tot * dis[:, None] + b2_ref[...]


def _row_spec():
    return pl.BlockSpec((BM, D), lambda i: (i, 0))


def _deg_spec():
    return pl.BlockSpec((NW, NP), lambda i: (0, 0))


def _agg_spec():
    return pl.BlockSpec((NC, BM, D), lambda i: (0, i, 0))


def _full_spec(shape):
    nd = len(shape)
    return pl.BlockSpec(shape, lambda i: (0,) * nd)


def _mm_scale(deg_parts, x, W):
    return pl.pallas_call(
        _mm_scale_body,
        grid=(G,),
        in_specs=[_deg_spec(), _row_spec(), _full_spec((D, D))],
        out_specs=_row_spec(),
        out_shape=jax.ShapeDtypeStruct((NP, D), jnp.float32),
    )(deg_parts, x, W)


def _layer2(deg_parts, agg, b1, W2):
    return pl.pallas_call(
        _layer2_body,
        grid=(G,),
        in_specs=[_deg_spec(), _agg_spec(),
                  _full_spec((1, D)), _full_spec((D, D))],
        out_specs=_row_spec(),
        out_shape=jax.ShapeDtypeStruct((NP, D), jnp.float32),
    )(deg_parts, agg, b1, W2)


def _final(deg_parts, agg, b2):
    return pl.pallas_call(
        _final_body,
        grid=(G,),
        in_specs=[_deg_spec(), _agg_spec(), _full_spec((1, D))],
        out_specs=_row_spec(),
        out_shape=jax.ShapeDtypeStruct((N, D), jnp.float32),
    )(deg_parts, agg, b2)


def kernel(x, edge_index, W1, b1, W2, b2):
    src = edge_index[0].astype(jnp.int32).reshape(NW, C, 1, K)
    dst = edge_index[1].astype(jnp.int32).reshape(NW, C, 1, K)
    sd = jnp.concatenate([src, dst], axis=2)
    dst_flat = edge_index[1].astype(jnp.int32).reshape(NW, EPW)
    zeros_deg = jnp.zeros((NP,), jnp.float32)
    zeros_row = jnp.zeros((TPN, D), jnp.float32)

    deg_parts = _deg_kernel(dst_flat, zeros_deg)
    y1 = _mm_scale(deg_parts, x, W1)
    agg1 = _scatter_kernel(y1, sd, zeros_row)
    y2 = _layer2(deg_parts, agg1, b1.reshape(1, D), W2)
    agg2 = _scatter_kernel(y2, sd, zeros_row)
    return _final(deg_parts, agg2, b2.reshape(1, D))
